# Initial kernel scaffold; baseline (speedup 1.0000x reference)
import jax
import jax.numpy as jnp
from jax.experimental import pallas as pl


def kernel(x_in, edge_index, edge_atts, w11, b11, w12, b12, root1, bias1, w21, b21, w22, b22, root2, bias2):
    raise NotImplementedError("write your pallas kernel here")


# speedup vs baseline: 3.9769x; 3.9769x over previous
import functools

import jax
import jax.numpy as jnp
from jax import lax
from jax.experimental import pallas as pl
from jax.experimental.pallas import tpu as pltpu
from jax.experimental.pallas import tpu_sc as plsc

N = 10000
E = 320000
NW = 32
PERW = E // NW
CH = 2000
NCH = PERW // CH

EB = 2560
EG = E // EB


def _sc_gather(table, idx):
    mesh = plsc.VectorSubcoreMesh(core_axis_name="c", subcore_axis_name="s")

    @functools.partial(
        pl.kernel, mesh=mesh,
        out_type=jax.ShapeDtypeStruct((E, 16), jnp.float32),
        compiler_params=pltpu.CompilerParams(use_tc_tiling_on_sc=False),
        scratch_types=[
            pltpu.VMEM((CH,), jnp.int32),
            pltpu.VMEM((CH, 16), jnp.float32),
            pltpu.SemaphoreType.DMA,
        ],
    )
    def gather_k(table_hbm, idx_hbm, out_hbm, idx_v, rows_v, sem):
        wid = lax.axis_index("s") * 2 + lax.axis_index("c")
        base = wid * PERW

        def body(ci, carry):
            off = base + ci * CH
            pltpu.sync_copy(idx_hbm.at[pl.ds(off, CH)], idx_v)
            pltpu.async_copy(table_hbm.at[idx_v], rows_v, sem).wait()
            pltpu.sync_copy(rows_v, out_hbm.at[pl.ds(off, CH)])
            return carry

        lax.fori_loop(0, NCH, body, 0)

    return gather_k(table, idx)


def _sc_scatter_add(vals, idx, zeros):
    mesh = plsc.VectorSubcoreMesh(core_axis_name="c", subcore_axis_name="s")

    @functools.partial(
        pl.kernel, mesh=mesh,
        out_type=jax.ShapeDtypeStruct((2, N, 16), jnp.float32),
        compiler_params=pltpu.CompilerParams(use_tc_tiling_on_sc=False),
        scratch_types=[
            pltpu.VMEM((CH,), jnp.int32),
            pltpu.VMEM((CH, 16), jnp.float32),
            pltpu.VMEM_SHARED((N, 16), jnp.float32),
        ],
    )
    def scatter_k(vals_hbm, idx_hbm, zeros_hbm, out_hbm, idx_v, val_v, acc_sh):
        cid = lax.axis_index("c")
        sid = lax.axis_index("s")

        @pl.when(sid == 0)
        def _init():
            pltpu.sync_copy(zeros_hbm, acc_sh)

        plsc.subcore_barrier()

        base = (sid * 2 + cid) * PERW

        def body(ci, carry):
            off = base + ci * CH
            pltpu.sync_copy(idx_hbm.at[pl.ds(off, CH)], idx_v)
            pltpu.sync_copy(vals_hbm.at[pl.ds(off, CH)], val_v)
            pltpu.sync_copy(val_v, acc_sh.at[idx_v], add=True)
            return carry

        lax.fori_loop(0, NCH, body, 0)
        plsc.subcore_barrier()

        rows = N // 16
        pltpu.sync_copy(acc_sh.at[pl.ds(sid * rows, rows)],
                        out_hbm.at[cid, pl.ds(sid * rows, rows)])

    return scatter_k(vals, idx, zeros)


def _edges_body(ea_ref, xj_ref, w1_ref, b1_ref, w2_ref, b2_ref, t_ref, r_ref,
                out_ref):
    h = jnp.dot(ea_ref[...], w1_ref[...], preferred_element_type=jnp.float32)
    h = jnp.maximum(h + b1_ref[...], 0.0)
    wm = jnp.dot(h, w2_ref[...], preferred_element_type=jnp.float32)
    wm = wm + b2_ref[...]
    xt = jnp.dot(xj_ref[...], t_ref[...], preferred_element_type=jnp.float32)
    out_ref[...] = jnp.dot(xt * wm, r_ref[...],
                           preferred_element_type=jnp.float32)


def _tc_edges(ea, xj, w1, b1, w2, b2, t, r):
    h1 = w1.shape[1]
    o1 = w2.shape[1]
    grid = (EG,)
    return pl.pallas_call(
        _edges_body,
        grid=grid,
        in_specs=[
            pl.BlockSpec((EB, 16), lambda i: (i, 0)),
            pl.BlockSpec((EB, 16), lambda i: (i, 0)),
            pl.BlockSpec((16, h1), lambda i: (0, 0)),
            pl.BlockSpec((1, h1), lambda i: (0, 0)),
            pl.BlockSpec((h1, o1), lambda i: (0, 0)),
            pl.BlockSpec((1, o1), lambda i: (0, 0)),
            pl.BlockSpec((16, o1), lambda i: (0, 0)),
            pl.BlockSpec((o1, 16), lambda i: (0, 0)),
        ],
        out_specs=pl.BlockSpec((EB, 16), lambda i: (i, 0)),
        out_shape=jax.ShapeDtypeStruct((E, 16), jnp.float32),
    )(ea, xj, w1, b1, w2, b2, t, r)


def _comb1_body(p_ref, x_ref, root_ref, bias_ref, out_ref):
    agg = p_ref[0] + p_ref[1]
    rt = jnp.dot(x_ref[...], root_ref[...], preferred_element_type=jnp.float32)
    out_ref[...] = jnp.maximum(agg + rt + bias_ref[...], 0.0)


def _tc_combine1(parts, x, root, bias):
    return pl.pallas_call(
        _comb1_body,
        out_shape=jax.ShapeDtypeStruct((N, 16), jnp.float32),
    )(parts, x, root, bias)


def _comb2_body(p_ref, x_ref, root_ref, bias_ref, out_ref):
    agg = p_ref[0] + p_ref[1]
    rt = jnp.dot(x_ref[...], root_ref[...], preferred_element_type=jnp.float32)
    y = agg[:, :8] + rt + bias_ref[...]
    m = jnp.max(y, axis=1, keepdims=True)
    lse = jnp.log(jnp.sum(jnp.exp(y - m), axis=1, keepdims=True)) + m
    out_ref[...] = y - lse


def _tc_combine2(parts, x, root, bias):
    return pl.pallas_call(
        _comb2_body,
        out_shape=jax.ShapeDtypeStruct((N, 8), jnp.float32),
    )(parts, x, root, bias)


def kernel(x_in, edge_index, edge_atts, w11, b11, w12, b12, root1, bias1,
           w21, b21, w22, b22, root2, bias2):
    src = edge_index[0]
    dst = edge_index[1]

    f32 = jnp.float32
    eye16 = jnp.eye(16, dtype=f32)
    t1 = jnp.repeat(eye16, 16, axis=1)
    r1 = jnp.tile(eye16, (16, 1))
    t2 = jnp.repeat(eye16, 8, axis=1)
    r2 = jnp.pad(jnp.tile(jnp.eye(8, dtype=f32), (16, 1)), ((0, 0), (0, 8)))

    zeros = jnp.zeros((N, 16), f32)

    xj1 = _sc_gather(x_in, src)
    msg1 = _tc_edges(edge_atts, xj1, w11, b11[None, :], w12, b12[None, :],
                     t1, r1)
    p1 = _sc_scatter_add(msg1, dst, zeros)
    x1 = _tc_combine1(p1, x_in, root1, bias1[None, :])

    xj2 = _sc_gather(x1, src)
    msg2 = _tc_edges(edge_atts, xj2, w21, b21[None, :], w22, b22[None, :],
                     t2, r2)
    p2 = _sc_scatter_add(msg2, dst, zeros)
    return _tc_combine2(p2, x1, root2, bias2[None, :])


# speedup vs baseline: 6.9817x; 1.7555x over previous
import functools

import jax
import jax.numpy as jnp
from jax import lax
from jax.experimental import pallas as pl
from jax.experimental.pallas import tpu as pltpu
from jax.experimental.pallas import tpu_sc as plsc

N = 10000
E = 320000
NW = 32
PERW = E // NW
CH = 2000
NCH = PERW // CH

EB = 2560
EG = E // EB


def _sc_gather(table, idx):
    mesh = plsc.VectorSubcoreMesh(core_axis_name="c", subcore_axis_name="s")

    @functools.partial(
        pl.kernel, mesh=mesh,
        out_type=jax.ShapeDtypeStruct((E, 16), jnp.float32),
        compiler_params=pltpu.CompilerParams(use_tc_tiling_on_sc=False),
        scratch_types=[
            pltpu.VMEM((CH,), jnp.int32),
            pltpu.VMEM((CH, 16), jnp.float32),
            pltpu.SemaphoreType.DMA,
        ],
    )
    def gather_k(table_hbm, idx_hbm, out_hbm, idx_v, rows_v, sem):
        wid = lax.axis_index("s") * 2 + lax.axis_index("c")
        base = wid * PERW

        def body(ci, carry):
            off = base + ci * CH
            pltpu.sync_copy(idx_hbm.at[pl.ds(off, CH)], idx_v)
            pltpu.async_copy(table_hbm.at[idx_v], rows_v, sem).wait()
            pltpu.sync_copy(rows_v, out_hbm.at[pl.ds(off, CH)])
            return carry

        lax.fori_loop(0, NCH, body, 0)

    return gather_k(table, idx)


def _sc_scatter_add(vals, idx, zeros):
    mesh = plsc.VectorSubcoreMesh(core_axis_name="c", subcore_axis_name="s")

    @functools.partial(
        pl.kernel, mesh=mesh,
        out_type=jax.ShapeDtypeStruct((2, N, 16), jnp.float32),
        compiler_params=pltpu.CompilerParams(use_tc_tiling_on_sc=False),
        scratch_types=[
            pltpu.VMEM((CH,), jnp.int32),
            pltpu.VMEM((CH, 16), jnp.float32),
            pltpu.VMEM_SHARED((N, 16), jnp.float32),
        ],
    )
    def scatter_k(vals_hbm, idx_hbm, zeros_hbm, out_hbm, idx_v, val_v, acc_sh):
        cid = lax.axis_index("c")
        sid = lax.axis_index("s")

        @pl.when(sid == 0)
        def _init():
            pltpu.sync_copy(zeros_hbm, acc_sh)

        plsc.subcore_barrier()

        base = (sid * 2 + cid) * PERW

        def body(ci, carry):
            off = base + ci * CH
            pltpu.sync_copy(idx_hbm.at[pl.ds(off, CH)], idx_v)
            pltpu.sync_copy(vals_hbm.at[pl.ds(off, CH)], val_v)
            pltpu.sync_copy(val_v, acc_sh.at[idx_v], add=True)
            return carry

        lax.fori_loop(0, NCH, body, 0)
        plsc.subcore_barrier()

        rows = N // 16
        pltpu.sync_copy(acc_sh.at[pl.ds(sid * rows, rows)],
                        out_hbm.at[cid, pl.ds(sid * rows, rows)])

    return scatter_k(vals, idx, zeros)


def _edges_body(ea_ref, xj_ref, w1_ref, b1_ref, w2_ref, b2_ref, t_ref, r_ref,
                out_ref):
    ea_p = ea_ref[...]
    xj_p = xj_ref[...]
    ea = jnp.concatenate([ea_p[:, 16 * j:16 * (j + 1)] for j in range(8)],
                         axis=0)
    xj = jnp.concatenate([xj_p[:, 16 * j:16 * (j + 1)] for j in range(8)],
                         axis=0)
    h = jnp.dot(ea, w1_ref[...], preferred_element_type=jnp.float32)
    h = jnp.maximum(h + b1_ref[...], 0.0)
    wm = jnp.dot(h, w2_ref[...], preferred_element_type=jnp.float32)
    wm = wm + b2_ref[...]
    xt = jnp.dot(xj, t_ref[...], preferred_element_type=jnp.float32)
    msg = jnp.dot(xt * wm, r_ref[...], preferred_element_type=jnp.float32)
    q = EB // 8
    out_ref[...] = jnp.concatenate([msg[q * j:q * (j + 1), :]
                                    for j in range(8)], axis=1)


def _tc_edges(ea_p, xj_p, w1, b1, w2, b2, t, r):
    h1 = w1.shape[1]
    o1 = w2.shape[1]
    grid = (EG,)
    return pl.pallas_call(
        _edges_body,
        grid=grid,
        in_specs=[
            pl.BlockSpec((EB // 8, 128), lambda i: (i, 0)),
            pl.BlockSpec((EB // 8, 128), lambda i: (i, 0)),
            pl.BlockSpec((16, h1), lambda i: (0, 0)),
            pl.BlockSpec((1, h1), lambda i: (0, 0)),
            pl.BlockSpec((h1, o1), lambda i: (0, 0)),
            pl.BlockSpec((1, o1), lambda i: (0, 0)),
            pl.BlockSpec((16, o1), lambda i: (0, 0)),
            pl.BlockSpec((o1, 16), lambda i: (0, 0)),
        ],
        out_specs=pl.BlockSpec((EB // 8, 128), lambda i: (i, 0)),
        out_shape=jax.ShapeDtypeStruct((E // 8, 128), jnp.float32),
    )(ea_p, xj_p, w1, b1, w2, b2, t, r)


def _comb1_body(p_ref, x_ref, root_ref, bias_ref, out_ref):
    agg = p_ref[0] + p_ref[1]
    rt = jnp.dot(x_ref[...], root_ref[...], preferred_element_type=jnp.float32)
    out_ref[...] = jnp.maximum(agg + rt + bias_ref[...], 0.0)


def _tc_combine1(parts, x, root, bias):
    return pl.pallas_call(
        _comb1_body,
        out_shape=jax.ShapeDtypeStruct((N, 16), jnp.float32),
    )(parts, x, root, bias)


def _comb2_body(p_ref, x_ref, root_ref, bias_ref, out_ref):
    agg = p_ref[0] + p_ref[1]
    rt = jnp.dot(x_ref[...], root_ref[...], preferred_element_type=jnp.float32)
    y = agg[:, :8] + rt + bias_ref[...]
    m = jnp.max(y, axis=1, keepdims=True)
    lse = jnp.log(jnp.sum(jnp.exp(y - m), axis=1, keepdims=True)) + m
    out_ref[...] = y - lse


def _tc_combine2(parts, x, root, bias):
    return pl.pallas_call(
        _comb2_body,
        out_shape=jax.ShapeDtypeStruct((N, 8), jnp.float32),
    )(parts, x, root, bias)


def kernel(x_in, edge_index, edge_atts, w11, b11, w12, b12, root1, bias1,
           w21, b21, w22, b22, root2, bias2):
    src = edge_index[0]
    dst = edge_index[1]

    f32 = jnp.float32
    eye16 = jnp.eye(16, dtype=f32)
    t1 = jnp.repeat(eye16, 16, axis=1)
    r1 = jnp.tile(eye16, (16, 1))
    t2 = jnp.repeat(eye16, 8, axis=1)
    r2 = jnp.pad(jnp.tile(jnp.eye(8, dtype=f32), (16, 1)), ((0, 0), (0, 8)))

    zeros = jnp.zeros((N, 16), f32)
    ea_p = edge_atts.reshape(E // 8, 128)

    xj1 = _sc_gather(x_in, src)
    msg1 = _tc_edges(ea_p, xj1.reshape(E // 8, 128), w11, b11[None, :],
                     w12, b12[None, :], t1, r1)
    p1 = _sc_scatter_add(msg1.reshape(E, 16), dst, zeros)
    x1 = _tc_combine1(p1, x_in, root1, bias1[None, :])

    xj2 = _sc_gather(x1, src)
    msg2 = _tc_edges(ea_p, xj2.reshape(E // 8, 128), w21, b21[None, :],
                     w22, b22[None, :], t2, r2)
    p2 = _sc_scatter_add(msg2.reshape(E, 16), dst, zeros)
    return _tc_combine2(p2, x1, root2, bias2[None, :])
